# baseline (speedup 1.0000x reference)
import functools

import jax
import jax.numpy as jnp
from jax import lax
from jax.experimental import pallas as pl
from jax.experimental.pallas import tpu as pltpu
from jax.experimental.pallas import tpu_sc as plsc

NC = 2
NS = 16
LANES = 16
EH = 16


def _pool_sums_sc(x, tab_lin, B, T):
    TH0 = 104
    TH1 = T - TH0
    NW = NC * NS
    BPW = B // NW
    GROUP = 64
    NG = BPW // GROUP
    NBUF = 4
    RS = GROUP // NBUF

    mesh = plsc.VectorSubcoreMesh(
        core_axis_name="c", subcore_axis_name="s",
        num_cores=NC, num_subcores=NS)

    @functools.partial(
        pl.kernel,
        out_type=jax.ShapeDtypeStruct((B, EH), jnp.float32),
        mesh=mesh,
        compiler_params=pltpu.CompilerParams(use_tc_tiling_on_sc=False),
        scratch_types=[
            pltpu.VMEM((2, GROUP, T), jnp.int32),
            pltpu.VMEM((NBUF, T, EH), jnp.float32),
            pltpu.VMEM((BPW, EH), jnp.float32),
            pltpu.SemaphoreType.DMA,
            pltpu.SemaphoreType.DMA,
            pltpu.SemaphoreType.DMA,
            pltpu.SemaphoreType.DMA,
            pltpu.SemaphoreType.DMA,
        ],
    )
    def pool(x_hbm, tab_hbm, out_hbm, idxv, bufs, outv,
             s0, s1, s2, s3, sidx):
        sems = (s0, s1, s2, s3)
        wid = lax.axis_index("s") * NC + lax.axis_index("c")
        xbase = wid * BPW
        obase = wid * BPW

        def row_descs(p, lr, bq):
            d0 = pltpu.make_async_copy(
                tab_hbm.at[idxv.at[p, lr, pl.ds(0, TH0)]],
                bufs.at[bq].at[pl.ds(0, TH0)], sems[bq])
            d1 = pltpu.make_async_copy(
                tab_hbm.at[idxv.at[p, lr, pl.ds(TH0, TH1)]],
                bufs.at[bq].at[pl.ds(TH0, TH1)], sems[bq])
            return d0, d1

        def fire(p, lr, bq):
            d0, d1 = row_descs(p, lr, bq)
            d0.start()
            d1.start()

        def drain(p, lr, bq):
            d0, d1 = row_descs(p, lr, bq)
            d0.wait()
            d1.wait()

        def accumulate(bq, row):
            zero = jnp.zeros((LANES,), jnp.float32)

            def body(j, carry):
                a0, a1 = carry
                j4 = j * 4
                a0 = a0 + bufs[bq, j4, pl.ds(0, LANES)]
                a1 = a1 + bufs[bq, j4 + 1, pl.ds(0, LANES)]
                a0 = a0 + bufs[bq, j4 + 2, pl.ds(0, LANES)]
                a1 = a1 + bufs[bq, j4 + 3, pl.ds(0, LANES)]
                return a0, a1

            a0, a1 = lax.fori_loop(0, T // 4, body, (zero, zero))
            outv[row, pl.ds(0, LANES)] = a0 + a1

        def idx_load_desc(g, p):
            return pltpu.make_async_copy(
                x_hbm.at[pl.ds(xbase + g * GROUP, GROUP)], idxv.at[p], sidx)

        idx_load_desc(0, 0).start()
        idx_load_desc(0, 0).wait()
        idx_load_desc(1, 1).start()

        for g in range(NG):
            p = g % 2
            if g > 0:
                idx_load_desc(g, p).wait()
            for bq in range(NBUF):
                fire(p, jnp.int32(bq), bq)

            def step(si, carry, p=p, g=g):
                for bq in range(NBUF):
                    lr = si * NBUF + bq
                    drain(p, lr, bq)
                    accumulate(bq, g * GROUP + lr)
                    fire(p, lr + NBUF, bq)
                return carry

            lax.fori_loop(0, RS - 1, step, 0)
            for bq in range(NBUF):
                lr = (RS - 1) * NBUF + bq
                drain(p, jnp.int32(lr), bq)
                accumulate(bq, g * GROUP + lr)
            if g + 2 < NG:
                idx_load_desc(g + 2, p).start()

        pltpu.sync_copy(outv, out_hbm.at[pl.ds(obase, BPW)])

    return pool(x, tab_lin)


def _transpose_half_tc(tabT, V, half, chunk=16384):
    nblk = (V + chunk - 1) // chunk
    vpad = nblk * chunk
    per = 128 // EH

    def body(t_ref, o_ref):
        t = t_ref[...]
        ii = lax.broadcasted_iota(jnp.int32, (EH, EH), 0)
        jj = lax.broadcasted_iota(jnp.int32, (EH, EH), 1)
        eye = (ii == jj).astype(jnp.float32)
        tt = lax.dot_general(
            t, eye, (((0,), (0,)), ((), ())),
            preferred_element_type=jnp.float32)
        tt3 = tt.reshape(chunk // per, per, EH)
        for a in range(per):
            o_ref[:, pl.ds(a * EH, EH)] = tt3[:, a, :]

    out = pl.pallas_call(
        body,
        grid=(nblk,),
        in_specs=[pl.BlockSpec((EH, chunk), lambda i, h=half: (h, i))],
        out_specs=pl.BlockSpec((chunk * EH // 128, 128), lambda i: (i, 0)),
        out_shape=jax.ShapeDtypeStruct((vpad * EH // 128, 128), jnp.float32),
    )(tabT)
    return out.reshape(vpad, EH)


def _linear_tc(p0, p1, W, b2, inv_t, B, C):
    BLK = 2048

    def body(p0_ref, p1_ref, w_ref, b_ref, o_ref):
        acc = jnp.dot(p0_ref[...], w_ref[0:EH, :],
                      preferred_element_type=jnp.float32)
        acc = acc + jnp.dot(p1_ref[...], w_ref[EH:2 * EH, :],
                            preferred_element_type=jnp.float32)
        o_ref[...] = acc * inv_t + b_ref[...]

    return pl.pallas_call(
        body,
        grid=(B // BLK,),
        in_specs=[
            pl.BlockSpec((BLK, EH), lambda i: (i, 0)),
            pl.BlockSpec((BLK, EH), lambda i: (i, 0)),
            pl.BlockSpec((2 * EH, C), lambda i: (0, 0)),
            pl.BlockSpec((1, C), lambda i: (0, 0)),
        ],
        out_specs=pl.BlockSpec((BLK, C), lambda i: (i, 0)),
        out_shape=jax.ShapeDtypeStruct((B, C), jnp.float32),
    )(p0, p1, W, b2)


def kernel(x, emb_table, W, b):
    B, T = x.shape
    V, E = emb_table.shape
    C = W.shape[1]
    tabT = emb_table.T
    t0 = _transpose_half_tc(tabT, V, 0)
    p0 = _pool_sums_sc(x, t0, B, T)
    t1 = _transpose_half_tc(tabT, V, 1)
    p1 = _pool_sums_sc(x, t1, B, T)
    return _linear_tc(p0, p1, W, b.reshape(1, C), 1.0 / T, B, C)


# speedup vs baseline: 2.0729x; 2.0729x over previous
import functools

import jax
import jax.numpy as jnp
from jax import lax
from jax.experimental import pallas as pl
from jax.experimental.pallas import tpu as pltpu
from jax.experimental.pallas import tpu_sc as plsc

NC = 2
NS = 16
LANES = 16


def _pool_sums_sc(x, emb_table, V, B, T, E):
    TH0 = 104
    TH1 = T - TH0
    NW = NC * NS
    BPW = B // NW
    GROUP = 64
    NG = BPW // GROUP
    NBUF = 4
    RS = GROUP // NBUF

    mesh = plsc.VectorSubcoreMesh(
        core_axis_name="c", subcore_axis_name="s",
        num_cores=NC, num_subcores=NS)

    @functools.partial(
        pl.kernel,
        out_type=jax.ShapeDtypeStruct((B, E), jnp.float32),
        mesh=mesh,
        compiler_params=pltpu.CompilerParams(use_tc_tiling_on_sc=False),
        scratch_types=[
            pltpu.VMEM((2, GROUP, T), jnp.int32),
            pltpu.VMEM((2, GROUP, T), jnp.int32),
            pltpu.VMEM((NBUF, T, E), jnp.float32),
            pltpu.VMEM((BPW, E), jnp.float32),
            pltpu.SemaphoreType.DMA,
            pltpu.SemaphoreType.DMA,
            pltpu.SemaphoreType.DMA,
            pltpu.SemaphoreType.DMA,
            pltpu.SemaphoreType.DMA,
        ],
    )
    def pool(x_hbm, tab_hbm, out_hbm, idxv, idxw, bufs, outv,
             s0, s1, s2, s3, sidx):

        sems = (s0, s1, s2, s3)
        wid = lax.axis_index("s") * NC + lax.axis_index("c")
        xbase = wid * BPW
        obase = wid * BPW

        def row_descs(p, lr, bq):
            d0 = pltpu.make_async_copy(
                tab_hbm.at[idxw.at[p, lr, pl.ds(0, TH0)]],
                bufs.at[bq].at[pl.ds(0, TH0)], sems[bq])
            d1 = pltpu.make_async_copy(
                tab_hbm.at[idxw.at[p, lr, pl.ds(TH0, TH1)]],
                bufs.at[bq].at[pl.ds(TH0, TH1)], sems[bq])
            return d0, d1

        def fire(p, lr, bq):
            d0, d1 = row_descs(p, lr, bq)
            d0.start()
            d1.start()

        def drain(p, lr, bq):
            d0, d1 = row_descs(p, lr, bq)
            d0.wait()
            d1.wait()

        def accumulate(bq, row):
            zero = jnp.zeros((LANES,), jnp.float32)

            def body(j, carry):
                a00, a01, a10, a11 = carry
                j4 = j * 4
                a00 = a00 + bufs[bq, j4, pl.ds(0, LANES)]
                a10 = a10 + bufs[bq, j4, pl.ds(LANES, LANES)]
                a01 = a01 + bufs[bq, j4 + 1, pl.ds(0, LANES)]
                a11 = a11 + bufs[bq, j4 + 1, pl.ds(LANES, LANES)]
                a00 = a00 + bufs[bq, j4 + 2, pl.ds(0, LANES)]
                a10 = a10 + bufs[bq, j4 + 2, pl.ds(LANES, LANES)]
                a01 = a01 + bufs[bq, j4 + 3, pl.ds(0, LANES)]
                a11 = a11 + bufs[bq, j4 + 3, pl.ds(LANES, LANES)]
                return a00, a01, a10, a11

            a00, a01, a10, a11 = lax.fori_loop(
                0, T // 4, body, (zero, zero, zero, zero))
            outv[row, pl.ds(0, LANES)] = a00 + a01
            outv[row, pl.ds(LANES, LANES)] = a10 + a11

        def remap_group(p):
            offs = [16 * k for k in range(T // 16)] + [T - 16]

            def trow(row, carry):
                for off in offs:
                    v = idxv[p, row, pl.ds(off, 16)]
                    w = ((v & jnp.int32(~16383))
                         | ((v & jnp.int32(4095)) << 2)
                         | ((v >> 12) & jnp.int32(3)))
                    idxw[p, row, pl.ds(off, 16)] = w
                return carry

            lax.fori_loop(0, GROUP, trow, 0)

        def idx_load_desc(g, p):
            return pltpu.make_async_copy(
                x_hbm.at[pl.ds(xbase + g * GROUP, GROUP)], idxv.at[p], sidx)

        def idx_load_start(g, p):
            idx_load_desc(g, p).start()

        def idx_load_wait(g, p):
            idx_load_desc(g, p).wait()

        idx_load_start(0, 0)
        idx_load_wait(0, 0)
        idx_load_start(1, 1)

        for g in range(NG):
            p = g % 2
            if g > 0:
                idx_load_wait(g, p)
            remap_group(p)
            for bq in range(NBUF):
                fire(p, jnp.int32(bq), bq)

            def step(si, carry, p=p, g=g):
                for bq in range(NBUF):
                    lr = si * NBUF + bq
                    drain(p, lr, bq)
                    accumulate(bq, g * GROUP + lr)
                    fire(p, lr + NBUF, bq)
                return carry

            lax.fori_loop(0, RS - 1, step, 0)
            for bq in range(NBUF):
                lr = (RS - 1) * NBUF + bq
                drain(p, jnp.int32(lr), bq)
                accumulate(bq, g * GROUP + lr)
            if g + 2 < NG:
                idx_load_start(g + 2, p)

        pltpu.sync_copy(outv, out_hbm.at[pl.ds(obase, BPW)])

    return pool(x, emb_table)


def _transpose_table_tc(tabT, V, E, chunk=16384):
    nblk = (V + chunk - 1) // chunk
    vpad = nblk * chunk

    def body(t_ref, o_ref):
        t = t_ref[...]
        q = chunk // 4
        m = jnp.concatenate(
            [t[:, a * q:(a + 1) * q] for a in range(4)], axis=0)
        o_ref[...] = m.T

    out = pl.pallas_call(
        body,
        grid=(nblk,),
        in_specs=[pl.BlockSpec((E, chunk), lambda i: (0, i))],
        out_specs=pl.BlockSpec((chunk * E // 128, 128), lambda i: (i, 0)),
        out_shape=jax.ShapeDtypeStruct((vpad * E // 128, 128), jnp.float32),
    )(tabT)
    return out.reshape(vpad, E), vpad


def _linear_tc(pooled_sums, W, b2, inv_t, B, E, C):
    BLK = 2048

    def body(p_ref, w_ref, b_ref, o_ref):
        o_ref[...] = (
            jnp.dot(p_ref[...], w_ref[...],
                    preferred_element_type=jnp.float32) * inv_t
            + b_ref[...])

    return pl.pallas_call(
        body,
        grid=(B // BLK,),
        in_specs=[
            pl.BlockSpec((BLK, E), lambda i: (i, 0)),
            pl.BlockSpec((E, C), lambda i: (0, 0)),
            pl.BlockSpec((1, C), lambda i: (0, 0)),
        ],
        out_specs=pl.BlockSpec((BLK, C), lambda i: (i, 0)),
        out_shape=jax.ShapeDtypeStruct((B, C), jnp.float32),
    )(pooled_sums, W, b2)


def kernel(x, emb_table, W, b):
    B, T = x.shape
    V, E = emb_table.shape
    C = W.shape[1]
    tab_lin, _ = _transpose_table_tc(emb_table.T, V, E)
    pooled_sums = _pool_sums_sc(x, tab_lin, V, B, T, E)
    return _linear_tc(pooled_sums, W, b.reshape(1, C), 1.0 / T, B, E, C)


# speedup vs baseline: 2.2382x; 1.0798x over previous
import functools

import jax
import jax.numpy as jnp
from jax import lax
from jax.experimental import pallas as pl
from jax.experimental.pallas import tpu as pltpu
from jax.experimental.pallas import tpu_sc as plsc

NC = 2
NS = 16
LANES = 16


def _pool_sums_sc(x, emb_table, V, B, T, E):
    TH0 = 104
    TH1 = T - TH0
    NW = NC * NS
    BPW = B // NW
    GROUP = 64
    NG = BPW // GROUP
    NBUF = 8
    RS = GROUP // NBUF

    mesh = plsc.VectorSubcoreMesh(
        core_axis_name="c", subcore_axis_name="s",
        num_cores=NC, num_subcores=NS)

    @functools.partial(
        pl.kernel,
        out_type=jax.ShapeDtypeStruct((B, E), jnp.float32),
        mesh=mesh,
        compiler_params=pltpu.CompilerParams(use_tc_tiling_on_sc=False),
        scratch_types=[
            pltpu.VMEM((2, GROUP, T), jnp.int32),
            pltpu.VMEM((2, GROUP, T), jnp.int32),
            pltpu.VMEM((NBUF, T, E), jnp.float32),
            pltpu.VMEM((BPW, E), jnp.float32),
            pltpu.SemaphoreType.DMA,
            pltpu.SemaphoreType.DMA,
            pltpu.SemaphoreType.DMA,
            pltpu.SemaphoreType.DMA,
            pltpu.SemaphoreType.DMA,
            pltpu.SemaphoreType.DMA,
            pltpu.SemaphoreType.DMA,
            pltpu.SemaphoreType.DMA,
            pltpu.SemaphoreType.DMA,
        ],
    )
    def pool(x_hbm, tab_hbm, out_hbm, idxv, idxw, bufs, outv,
             s0, s1, s2, s3, s4, s5, s6, s7, sidx):

        sems = (s0, s1, s2, s3, s4, s5, s6, s7)
        wid = lax.axis_index("s") * NC + lax.axis_index("c")
        xbase = wid * BPW
        obase = wid * BPW

        def row_descs(p, lr, bq):
            d0 = pltpu.make_async_copy(
                tab_hbm.at[idxw.at[p, lr, pl.ds(0, TH0)]],
                bufs.at[bq].at[pl.ds(0, TH0)], sems[bq])
            d1 = pltpu.make_async_copy(
                tab_hbm.at[idxw.at[p, lr, pl.ds(TH0, TH1)]],
                bufs.at[bq].at[pl.ds(TH0, TH1)], sems[bq])
            return d0, d1

        def fire(p, lr, bq):
            d0, d1 = row_descs(p, lr, bq)
            d0.start()
            d1.start()

        def drain(p, lr, bq):
            d0, d1 = row_descs(p, lr, bq)
            d0.wait()
            d1.wait()

        def accumulate(bq, row):
            zero = jnp.zeros((LANES,), jnp.float32)

            def body(j, carry):
                a00, a01, a10, a11 = carry
                j4 = j * 4
                a00 = a00 + bufs[bq, j4, pl.ds(0, LANES)]
                a10 = a10 + bufs[bq, j4, pl.ds(LANES, LANES)]
                a01 = a01 + bufs[bq, j4 + 1, pl.ds(0, LANES)]
                a11 = a11 + bufs[bq, j4 + 1, pl.ds(LANES, LANES)]
                a00 = a00 + bufs[bq, j4 + 2, pl.ds(0, LANES)]
                a10 = a10 + bufs[bq, j4 + 2, pl.ds(LANES, LANES)]
                a01 = a01 + bufs[bq, j4 + 3, pl.ds(0, LANES)]
                a11 = a11 + bufs[bq, j4 + 3, pl.ds(LANES, LANES)]
                return a00, a01, a10, a11

            a00, a01, a10, a11 = lax.fori_loop(
                0, T // 4, body, (zero, zero, zero, zero))
            outv[row, pl.ds(0, LANES)] = a00 + a01
            outv[row, pl.ds(LANES, LANES)] = a10 + a11

        def remap_group(p):
            offs = [16 * k for k in range(T // 16)] + [T - 16]

            def trow(row, carry):
                for off in offs:
                    v = idxv[p, row, pl.ds(off, 16)]
                    w = ((v & jnp.int32(~16383))
                         | ((v & jnp.int32(4095)) << 2)
                         | ((v >> 12) & jnp.int32(3)))
                    idxw[p, row, pl.ds(off, 16)] = w
                return carry

            lax.fori_loop(0, GROUP, trow, 0)

        def idx_load_desc(g, p):
            return pltpu.make_async_copy(
                x_hbm.at[pl.ds(xbase + g * GROUP, GROUP)], idxv.at[p], sidx)

        def idx_load_start(g, p):
            idx_load_desc(g, p).start()

        def idx_load_wait(g, p):
            idx_load_desc(g, p).wait()

        idx_load_start(0, 0)
        idx_load_wait(0, 0)
        idx_load_start(1, 1)

        for g in range(NG):
            p = g % 2
            if g > 0:
                idx_load_wait(g, p)
            remap_group(p)
            for bq in range(NBUF):
                fire(p, jnp.int32(bq), bq)

            def step(si, carry, p=p, g=g):
                for bq in range(NBUF):
                    lr = si * NBUF + bq
                    drain(p, lr, bq)
                    accumulate(bq, g * GROUP + lr)
                    fire(p, lr + NBUF, bq)
                return carry

            lax.fori_loop(0, RS - 1, step, 0)
            for bq in range(NBUF):
                lr = (RS - 1) * NBUF + bq
                drain(p, jnp.int32(lr), bq)
                accumulate(bq, g * GROUP + lr)
            if g + 2 < NG:
                idx_load_start(g + 2, p)

        pltpu.sync_copy(outv, out_hbm.at[pl.ds(obase, BPW)])

    return pool(x, emb_table)


def _transpose_table_tc(tabT, V, E, chunk=16384):
    nblk = (V + chunk - 1) // chunk
    vpad = nblk * chunk

    def body(t_ref, o_ref):
        t = t_ref[...]
        q = chunk // 4
        m = jnp.concatenate(
            [t[:, a * q:(a + 1) * q] for a in range(4)], axis=0)
        o_ref[...] = m.T

    out = pl.pallas_call(
        body,
        grid=(nblk,),
        in_specs=[pl.BlockSpec((E, chunk), lambda i: (0, i))],
        out_specs=pl.BlockSpec((chunk * E // 128, 128), lambda i: (i, 0)),
        out_shape=jax.ShapeDtypeStruct((vpad * E // 128, 128), jnp.float32),
    )(tabT)
    return out.reshape(vpad, E), vpad


def _linear_tc(pooled_sums, W, b2, inv_t, B, E, C):
    BLK = 2048

    def body(p_ref, w_ref, b_ref, o_ref):
        o_ref[...] = (
            jnp.dot(p_ref[...], w_ref[...],
                    preferred_element_type=jnp.float32) * inv_t
            + b_ref[...])

    return pl.pallas_call(
        body,
        grid=(B // BLK,),
        in_specs=[
            pl.BlockSpec((BLK, E), lambda i: (i, 0)),
            pl.BlockSpec((E, C), lambda i: (0, 0)),
            pl.BlockSpec((1, C), lambda i: (0, 0)),
        ],
        out_specs=pl.BlockSpec((BLK, C), lambda i: (i, 0)),
        out_shape=jax.ShapeDtypeStruct((B, C), jnp.float32),
    )(pooled_sums, W, b2)


def kernel(x, emb_table, W, b):
    B, T = x.shape
    V, E = emb_table.shape
    C = W.shape[1]
    tab_lin, _ = _transpose_table_tc(emb_table.T, V, E)
    pooled_sums = _pool_sums_sc(x, tab_lin, V, B, T, E)
    return _linear_tc(pooled_sums, W, b.reshape(1, C), 1.0 / T, B, E, C)
